# baseline (speedup 1.0000x reference)
import jax
import jax.numpy as jnp
from jax import lax
from jax.experimental import pallas as pl
from jax.experimental.pallas import tpu as pltpu
from jax.experimental.pallas import tpu_sc as plsc

_ROWS = 4096
_COLS = 16384
_NSEL = 64

_NC = 2
_NS = 16
_LANES = 16
_NW = _NC * _NS
_RPW = _NSEL // _NW


def _copy_body(d_ref, o_ref):
    o_ref[...] = d_ref[...]


def _copy(data):
    block_rows = 128
    return pl.pallas_call(
        _copy_body,
        grid=(_ROWS // block_rows,),
        in_specs=[pl.BlockSpec((block_rows, _COLS), lambda i: (i, 0))],
        out_specs=pl.BlockSpec((block_rows, _COLS), lambda i: (i, 0)),
        out_shape=jax.ShapeDtypeStruct((_ROWS, _COLS), jnp.float32),
    )(data)


def _gather_body(sel_hbm, data_hbm, out_hbm, sel_v, rows_v, sem):
    wid = lax.axis_index("s") * _NC + lax.axis_index("c")
    pltpu.sync_copy(sel_hbm, sel_v)
    idx = sel_v.at[wid]
    pltpu.async_copy(data_hbm.at[idx], rows_v, sem).wait()

    def _neg(i, carry):
        for r in range(_RPW):
            rows_v[r, pl.ds(i * _LANES, _LANES)] = -rows_v[r, pl.ds(i * _LANES, _LANES)]
        return carry

    lax.fori_loop(0, _COLS // _LANES, _neg, 0)
    pltpu.sync_copy(rows_v, out_hbm.at[pl.ds(wid * _RPW, _RPW)])


def _gather_neg_rows(sel, data):
    mesh = plsc.VectorSubcoreMesh(core_axis_name="c", subcore_axis_name="s")
    return pl.kernel(
        _gather_body,
        out_type=jax.ShapeDtypeStruct((_NSEL, _COLS), jnp.float32),
        mesh=mesh,
        scratch_types=[
            pltpu.VMEM((_NW, _RPW), jnp.int32),
            pltpu.VMEM((_RPW, _COLS), jnp.float32),
            pltpu.SemaphoreType.DMA,
        ],
        compiler_params=pltpu.CompilerParams(needs_layout_passes=False),
    )(sel.reshape(_NW, _RPW), data)


def _scatter_body(sel_ref, alias_ref, neg_ref, o_ref, sem):
    del alias_ref
    copies = [
        pltpu.make_async_copy(neg_ref.at[j], o_ref.at[sel_ref[j]], sem)
        for j in range(_NSEL)
    ]
    for c in copies:
        c.start()
    for c in copies:
        c.wait()


def _scatter(sel, out1, negrows):
    return pl.pallas_call(
        _scatter_body,
        in_specs=[
            pl.BlockSpec(memory_space=pltpu.MemorySpace.SMEM),
            pl.BlockSpec(memory_space=pltpu.MemorySpace.HBM),
            pl.BlockSpec(memory_space=pltpu.MemorySpace.VMEM),
        ],
        out_specs=pl.BlockSpec(memory_space=pltpu.MemorySpace.HBM),
        out_shape=jax.ShapeDtypeStruct((_ROWS, _COLS), jnp.float32),
        scratch_shapes=[pltpu.SemaphoreType.DMA],
        input_output_aliases={1: 0},
    )(sel, out1, negrows)


def kernel(data, selection):
    sel = selection.astype(jnp.int32)
    negrows = _gather_neg_rows(sel, data)
    out1 = _copy(data)
    return _scatter(sel, out1, negrows)


# speedup vs baseline: 1.1316x; 1.1316x over previous
import functools

import jax
import jax.numpy as jnp
from jax.experimental import pallas as pl
from jax.experimental.pallas import tpu as pltpu

_ROWS = 4096
_COLS = 16384
_NSEL = 64


def _flip_body(block_rows, sel_ref, d_ref, o_ref):
    i = pl.program_id(0)
    rows = jax.lax.broadcasted_iota(jnp.int32, (block_rows, _NSEL), 0) + i * block_rows
    match = (rows == sel_ref[...]).any(axis=1, keepdims=True)
    sign = jnp.where(match, -1.0, 1.0).astype(jnp.float32)
    o_ref[...] = d_ref[...] * sign


def kernel(data, selection):
    sel = selection.astype(jnp.int32).reshape(1, _NSEL)
    block_rows = 64
    grid = (_ROWS // block_rows,)
    return pl.pallas_call(
        functools.partial(_flip_body, block_rows),
        grid=grid,
        in_specs=[
            pl.BlockSpec((1, _NSEL), lambda i: (0, 0)),
            pl.BlockSpec((block_rows, _COLS), lambda i: (i, 0)),
        ],
        out_specs=pl.BlockSpec((block_rows, _COLS), lambda i: (i, 0)),
        out_shape=jax.ShapeDtypeStruct((_ROWS, _COLS), jnp.float32),
    )(sel, data)


# speedup vs baseline: 1.1446x; 1.0115x over previous
import functools

import jax
import jax.numpy as jnp
from jax.experimental import pallas as pl
from jax.experimental.pallas import tpu as pltpu

_ROWS = 4096
_COLS = 16384
_NSEL = 64


def _flip_body(block_rows, sel_ref, d_ref, o_ref):
    i = pl.program_id(0)
    rows = jax.lax.broadcasted_iota(jnp.int32, (block_rows, _NSEL), 0) + i * block_rows
    match = (rows == sel_ref[...]).any(axis=1, keepdims=True)
    sign = jnp.where(match, -1.0, 1.0).astype(jnp.float32)
    o_ref[...] = d_ref[...] * sign


def kernel(data, selection):
    sel = selection.astype(jnp.int32).reshape(1, _NSEL)
    block_rows = 256
    block_cols = 8192
    grid = (_ROWS // block_rows, _COLS // block_cols)
    return pl.pallas_call(
        functools.partial(_flip_body, block_rows),
        grid=grid,
        in_specs=[
            pl.BlockSpec((1, _NSEL), lambda i, j: (0, 0)),
            pl.BlockSpec((block_rows, block_cols), lambda i, j: (i, j)),
        ],
        out_specs=pl.BlockSpec((block_rows, block_cols), lambda i, j: (i, j)),
        out_shape=jax.ShapeDtypeStruct((_ROWS, _COLS), jnp.float32),
    )(sel, data)


# speedup vs baseline: 1.1467x; 1.0018x over previous
import functools

import jax
import jax.numpy as jnp
from jax.experimental import pallas as pl
from jax.experimental.pallas import tpu as pltpu

_ROWS = 4096
_COLS = 16384
_NSEL = 64


def _flip_body(block_rows, sel_ref, d_ref, o_ref):
    i = pl.program_id(0)
    rows = jax.lax.broadcasted_iota(jnp.int32, (block_rows, _NSEL), 0) + i * block_rows
    match = (rows == sel_ref[...]).any(axis=1, keepdims=True)
    sign = jnp.where(match, -1.0, 1.0).astype(jnp.float32)
    o_ref[...] = d_ref[...] * sign


def kernel(data, selection):
    sel = selection.astype(jnp.int32).reshape(1, _NSEL)
    block_rows = 128
    grid = (_ROWS // block_rows,)
    return pl.pallas_call(
        functools.partial(_flip_body, block_rows),
        grid=grid,
        in_specs=[
            pl.BlockSpec((1, _NSEL), lambda i: (0, 0)),
            pl.BlockSpec((block_rows, _COLS), lambda i: (i, 0)),
        ],
        out_specs=pl.BlockSpec((block_rows, _COLS), lambda i: (i, 0)),
        out_shape=jax.ShapeDtypeStruct((_ROWS, _COLS), jnp.float32),
    )(sel, data)
